# baseline (speedup 1.0000x reference)
import functools

import jax
import jax.numpy as jnp
from jax import lax
from jax.experimental import pallas as pl
from jax.experimental.pallas import tpu as pltpu
from jax.experimental.pallas import tpu_sc as plsc

LANES = 16
CHUNK = 64
TBLOCK = 2048


def _vec_sqrt(x):
    xg = jnp.maximum(x, jnp.float32(1e-35))
    i = lax.bitcast_convert_type(xg, jnp.int32)
    i = jnp.int32(0x5F3759DF) - lax.shift_right_logical(i, jnp.int32(1))
    y = lax.bitcast_convert_type(i, jnp.float32)
    half = jnp.float32(0.5) * xg
    for _ in range(3):
        y = y * (jnp.float32(1.5) - half * y * y)
    return x * y


def _transpose_pack(ent_t):
    D, E = ent_t.shape
    nb = -(-E // TBLOCK)
    hb = TBLOCK // 2

    def body(in_ref, out_ref):
        xt = jnp.transpose(in_ref[...])
        out_ref[...] = jnp.concatenate([xt[:hb], xt[hb:]], axis=1)

    return pl.pallas_call(
        body,
        grid=(nb,),
        in_specs=[pl.BlockSpec((D, TBLOCK), lambda i: (0, i))],
        out_specs=pl.BlockSpec((hb, 2 * D), lambda i: (i, 0)),
        out_shape=jax.ShapeDtypeStruct((nb * hb, 2 * D), jnp.float32),
    )(ent_t)


def _make_transe(B, D):
    info = plsc.get_sparse_core_info()
    NC, NS = info.num_cores, info.num_subcores
    NW = NC * NS
    per_w = B // NW
    n_chunks = per_w // CHUNK
    D2 = 2 * D
    assert per_w % CHUNK == 0 and D % LANES == 0

    mesh = plsc.VectorSubcoreMesh(core_axis_name="c", subcore_axis_name="s")

    row_buf = pltpu.VMEM((CHUNK, D2), jnp.float32)
    idx_buf = pltpu.VMEM((per_w,), jnp.int32)

    @functools.partial(
        pl.kernel,
        mesh=mesh,
        compiler_params=pltpu.CompilerParams(needs_layout_passes=False),
        out_type=(
            jax.ShapeDtypeStruct((B,), jnp.float32),
            jax.ShapeDtypeStruct((B,), jnp.float32),
        ),
        scratch_types=[
            idx_buf, idx_buf, idx_buf, idx_buf, idx_buf,
            idx_buf, idx_buf, idx_buf, idx_buf, idx_buf,
            row_buf, row_buf, row_buf, row_buf, row_buf,
            row_buf, row_buf, row_buf, row_buf, row_buf,
            pltpu.VMEM((per_w,), jnp.float32),
            pltpu.VMEM((per_w,), jnp.float32),
            pltpu.SemaphoreType.DMA,
            pltpu.SemaphoreType.DMA,
            pltpu.SemaphoreType.DMA,
        ],
    )
    def transe(ph_idx, pt_idx, nh_idx, nt_idx, r_idx, ent2, rel2,
               pos_out, neg_out,
               oph, opt, onh, ont, orl,
               tph, tpt, tnh, tnt, trl,
               ph0, pt0, nh0, nt0, rr0,
               ph1, pt1, nh1, nt1, rr1,
               po, no, sem_i, sem0, sem1):
        wid = lax.axis_index("s") * NC + lax.axis_index("c")
        base_w = wid * per_w
        offs_b = (oph, opt, onh, ont, orl)
        tids = (tph, tpt, tnh, tnt, trl)
        bufs = ((ph0, pt0, nh0, nt0, rr0), (ph1, pt1, nh1, nt1, rr1))
        sems = (sem0, sem1)

        idx_cps = [
            pltpu.async_copy(src.at[pl.ds(base_w, per_w)], dst, sem_i)
            for src, dst in zip(
                (ph_idx, pt_idx, nh_idx, nt_idx, r_idx), offs_b)
        ]
        for cp in idx_cps:
            cp.wait()

        hb_shift = jnp.int32(TBLOCK.bit_length() - 2)
        hb_mask = jnp.int32(TBLOCK // 2 - 1)
        dd = jnp.int32(D)

        def map_body(i, _):
            sl = pl.ds(i * LANES, LANES)
            for off, tid in zip(offs_b[:4], tids[:4]):
                e = off[sl]
                b = lax.shift_right_logical(e, hb_shift + 1)
                h = lax.shift_right_logical(e, hb_shift) & jnp.int32(1)
                tid[sl] = lax.shift_left(b, hb_shift) + (e & hb_mask)
                off[sl] = h * dd
            e = orl[sl]
            trl[sl] = lax.shift_right_logical(e, jnp.int32(1))
            orl[sl] = (e & jnp.int32(1)) * dd
            return 0

        lax.fori_loop(0, per_w // LANES, map_body, 0)

        def fire(c, par):
            sl = pl.ds(c * CHUNK, CHUNK)
            sem = sems[par]
            cps = []
            for tid, dst in zip(tids[:4], bufs[par][:4]):
                cps.append(pltpu.async_copy(ent2.at[tid.at[sl]], dst, sem))
            cps.append(pltpu.async_copy(rel2.at[trl.at[sl]], bufs[par][4], sem))
            return cps

        lane_ids = lax.iota(jnp.int32, LANES)
        in_flight = fire(0, 0)
        for c in range(n_chunks):
            par = c & 1
            for cp in in_flight:
                cp.wait()
            if c + 1 < n_chunks:
                in_flight = fire(c + 1, 1 - par)
            bset = bufs[par]
            out0 = c * CHUNK

            def group_body(g, _):
                row0 = g * LANES
                pvs = [off[pl.ds(out0 + row0, LANES)] for off in offs_b]
                pvec = jnp.zeros((LANES,), jnp.float32)
                nvec = jnp.zeros((LANES,), jnp.float32)
                for j in range(LANES):
                    r = row0 + j
                    offs = [pv[j] for pv in pvs]
                    pacc = jnp.zeros((LANES,), jnp.float32)
                    nacc = jnp.zeros((LANES,), jnp.float32)
                    for d in range(D // LANES):
                        hv = bset[0][r, pl.ds(offs[0] + d * LANES, LANES)]
                        tv = bset[1][r, pl.ds(offs[1] + d * LANES, LANES)]
                        nhv = bset[2][r, pl.ds(offs[2] + d * LANES, LANES)]
                        ntv = bset[3][r, pl.ds(offs[3] + d * LANES, LANES)]
                        rv = bset[4][r, pl.ds(offs[4] + d * LANES, LANES)]
                        pd = hv + rv - tv
                        nd = nhv + rv - ntv
                        pacc = pacc + pd * pd
                        nacc = nacc + nd * nd
                    jmask = lane_ids == j
                    pvec = jnp.where(jmask, jnp.sum(pacc), pvec)
                    nvec = jnp.where(jmask, jnp.sum(nacc), nvec)
                po[pl.ds(out0 + row0, LANES)] = _vec_sqrt(pvec)
                no[pl.ds(out0 + row0, LANES)] = _vec_sqrt(nvec)
                return 0

            lax.fori_loop(0, CHUNK // LANES, group_body, 0)

        pltpu.sync_copy(po, pos_out.at[pl.ds(base_w, per_w)])
        pltpu.sync_copy(no, neg_out.at[pl.ds(base_w, per_w)])

    return transe


def kernel(pos_edge_index, edge_type, neg_edge_index, entity_embeddings,
           relation_embeddings):
    B = pos_edge_index.shape[1]
    E, D = entity_embeddings.shape
    R = relation_embeddings.shape[0]
    ent2 = _transpose_pack(entity_embeddings.T)
    rel2 = relation_embeddings.reshape(R // 2, 2 * D)
    fn = _make_transe(B, D)
    return fn(pos_edge_index[0], pos_edge_index[1],
              neg_edge_index[0], neg_edge_index[1], edge_type, ent2, rel2)


# speedup vs baseline: 1.5991x; 1.5991x over previous
import functools

import jax
import jax.numpy as jnp
from jax import lax
from jax.experimental import pallas as pl
from jax.experimental.pallas import tpu as pltpu
from jax.experimental.pallas import tpu_sc as plsc

LANES = 16
CHUNK = 64
TBLOCK = 8192


def _vec_sqrt(x):
    xg = jnp.maximum(x, jnp.float32(1e-35))
    i = lax.bitcast_convert_type(xg, jnp.int32)
    i = jnp.int32(0x5F3759DF) - lax.shift_right_logical(i, jnp.int32(1))
    y = lax.bitcast_convert_type(i, jnp.float32)
    half = jnp.float32(0.5) * xg
    for _ in range(3):
        y = y * (jnp.float32(1.5) - half * y * y)
    return x * y


def _transpose_pack(ent_t):
    D, E = ent_t.shape
    nb = -(-E // TBLOCK)
    hb = TBLOCK // 2

    def body(in_ref, out_ref):
        lanes = jnp.arange(D, dtype=jnp.int32)
        ident = (lanes[:, None] == lanes[None, :]).astype(jnp.float32)
        xt = lax.dot_general(
            in_ref[...], ident, (((0,), (0,)), ((), ())),
            preferred_element_type=jnp.float32)
        out_ref[...] = jnp.concatenate([xt[:hb], xt[hb:]], axis=1)

    return pl.pallas_call(
        body,
        grid=(nb,),
        in_specs=[pl.BlockSpec((D, TBLOCK), lambda i: (0, i))],
        out_specs=pl.BlockSpec((hb, 2 * D), lambda i: (i, 0)),
        out_shape=jax.ShapeDtypeStruct((nb * hb, 2 * D), jnp.float32),
    )(ent_t)


def _make_transe(B, D):
    info = plsc.get_sparse_core_info()
    NC, NS = info.num_cores, info.num_subcores
    NW = NC * NS
    per_w = B // NW
    n_chunks = per_w // CHUNK
    D2 = 2 * D
    assert per_w % CHUNK == 0 and D % LANES == 0

    mesh = plsc.VectorSubcoreMesh(core_axis_name="c", subcore_axis_name="s")

    row_buf = pltpu.VMEM((CHUNK, D2), jnp.float32)
    idx_buf = pltpu.VMEM((per_w,), jnp.int32)

    @functools.partial(
        pl.kernel,
        mesh=mesh,
        compiler_params=pltpu.CompilerParams(needs_layout_passes=False),
        out_type=(
            jax.ShapeDtypeStruct((B,), jnp.float32),
            jax.ShapeDtypeStruct((B,), jnp.float32),
        ),
        scratch_types=[
            idx_buf, idx_buf, idx_buf, idx_buf, idx_buf,
            idx_buf, idx_buf, idx_buf, idx_buf, idx_buf,
            row_buf, row_buf, row_buf, row_buf, row_buf,
            row_buf, row_buf, row_buf, row_buf, row_buf,
            pltpu.VMEM((per_w,), jnp.float32),
            pltpu.VMEM((per_w,), jnp.float32),
            pltpu.SemaphoreType.DMA,
            pltpu.SemaphoreType.DMA,
            pltpu.SemaphoreType.DMA,
        ],
    )
    def transe(ph_idx, pt_idx, nh_idx, nt_idx, r_idx, ent2, rel2,
               pos_out, neg_out,
               oph, opt, onh, ont, orl,
               tph, tpt, tnh, tnt, trl,
               ph0, pt0, nh0, nt0, rr0,
               ph1, pt1, nh1, nt1, rr1,
               po, no, sem_i, sem0, sem1):
        wid = lax.axis_index("s") * NC + lax.axis_index("c")
        base_w = wid * per_w
        offs_b = (oph, opt, onh, ont, orl)
        tids = (tph, tpt, tnh, tnt, trl)
        bufs = ((ph0, pt0, nh0, nt0, rr0), (ph1, pt1, nh1, nt1, rr1))
        sems = (sem0, sem1)

        idx_cps = [
            pltpu.async_copy(src.at[pl.ds(base_w, per_w)], dst, sem_i)
            for src, dst in zip(
                (ph_idx, pt_idx, nh_idx, nt_idx, r_idx), offs_b)
        ]
        for cp in idx_cps:
            cp.wait()

        hb_shift = jnp.int32(TBLOCK.bit_length() - 2)
        hb_mask = jnp.int32(TBLOCK // 2 - 1)
        dd = jnp.int32(D)

        def map_body(i, _):
            sl = pl.ds(i * LANES, LANES)
            for off, tid in zip(offs_b[:4], tids[:4]):
                e = off[sl]
                b = lax.shift_right_logical(e, hb_shift + 1)
                h = lax.shift_right_logical(e, hb_shift) & jnp.int32(1)
                tid[sl] = lax.shift_left(b, hb_shift) + (e & hb_mask)
                off[sl] = h * dd
            e = orl[sl]
            trl[sl] = lax.shift_right_logical(e, jnp.int32(1))
            orl[sl] = (e & jnp.int32(1)) * dd
            return 0

        lax.fori_loop(0, per_w // LANES, map_body, 0)

        def fire(c, par):
            sl = pl.ds(c * CHUNK, CHUNK)
            sem = sems[par]
            cps = []
            for tid, dst in zip(tids[:4], bufs[par][:4]):
                cps.append(pltpu.async_copy(ent2.at[tid.at[sl]], dst, sem))
            cps.append(pltpu.async_copy(rel2.at[trl.at[sl]], bufs[par][4], sem))
            return cps

        lane_ids = lax.iota(jnp.int32, LANES)
        in_flight = fire(0, 0)
        for c in range(n_chunks):
            par = c & 1
            for cp in in_flight:
                cp.wait()
            if c + 1 < n_chunks:
                in_flight = fire(c + 1, 1 - par)
            bset = bufs[par]
            out0 = c * CHUNK

            def group_body(g, _):
                row0 = g * LANES
                pvs = [off[pl.ds(out0 + row0, LANES)] for off in offs_b]
                pvec = jnp.zeros((LANES,), jnp.float32)
                nvec = jnp.zeros((LANES,), jnp.float32)
                for j in range(LANES):
                    r = row0 + j
                    offs = [pv[j] for pv in pvs]
                    pacc = jnp.zeros((LANES,), jnp.float32)
                    nacc = jnp.zeros((LANES,), jnp.float32)
                    for d in range(D // LANES):
                        hv = bset[0][r, pl.ds(offs[0] + d * LANES, LANES)]
                        tv = bset[1][r, pl.ds(offs[1] + d * LANES, LANES)]
                        nhv = bset[2][r, pl.ds(offs[2] + d * LANES, LANES)]
                        ntv = bset[3][r, pl.ds(offs[3] + d * LANES, LANES)]
                        rv = bset[4][r, pl.ds(offs[4] + d * LANES, LANES)]
                        pd = hv + rv - tv
                        nd = nhv + rv - ntv
                        pacc = pacc + pd * pd
                        nacc = nacc + nd * nd
                    jmask = lane_ids == j
                    pvec = jnp.where(jmask, jnp.sum(pacc), pvec)
                    nvec = jnp.where(jmask, jnp.sum(nacc), nvec)
                po[pl.ds(out0 + row0, LANES)] = _vec_sqrt(pvec)
                no[pl.ds(out0 + row0, LANES)] = _vec_sqrt(nvec)
                return 0

            lax.fori_loop(0, CHUNK // LANES, group_body, 0)

        pltpu.sync_copy(po, pos_out.at[pl.ds(base_w, per_w)])
        pltpu.sync_copy(no, neg_out.at[pl.ds(base_w, per_w)])

    return transe


def kernel(pos_edge_index, edge_type, neg_edge_index, entity_embeddings,
           relation_embeddings):
    B = pos_edge_index.shape[1]
    E, D = entity_embeddings.shape
    R = relation_embeddings.shape[0]
    ent2 = _transpose_pack(entity_embeddings.T)
    rel2 = relation_embeddings.reshape(R // 2, 2 * D)
    fn = _make_transe(B, D)
    return fn(pos_edge_index[0], pos_edge_index[1],
              neg_edge_index[0], neg_edge_index[1], edge_type, ent2, rel2)


# speedup vs baseline: 1.7862x; 1.1170x over previous
import functools

import jax
import jax.numpy as jnp
from jax import lax
from jax.experimental import pallas as pl
from jax.experimental.pallas import tpu as pltpu
from jax.experimental.pallas import tpu_sc as plsc

LANES = 16
CHUNK = 64
TBLOCK = 16384


def _vec_sqrt(x):
    xg = jnp.maximum(x, jnp.float32(1e-35))
    i = lax.bitcast_convert_type(xg, jnp.int32)
    i = jnp.int32(0x5F3759DF) - lax.shift_right_logical(i, jnp.int32(1))
    y = lax.bitcast_convert_type(i, jnp.float32)
    half = jnp.float32(0.5) * xg
    for _ in range(3):
        y = y * (jnp.float32(1.5) - half * y * y)
    return x * y


def _transpose_pack(ent_t):
    D, E = ent_t.shape
    nb = -(-E // TBLOCK)
    hb = TBLOCK // 2

    def body(in_ref, out_ref):
        lanes = jnp.arange(D, dtype=jnp.int32)
        ident = (lanes[:, None] == lanes[None, :]).astype(jnp.float32)
        xt = lax.dot_general(
            in_ref[...], ident, (((0,), (0,)), ((), ())),
            preferred_element_type=jnp.float32)
        out_ref[...] = jnp.concatenate([xt[:hb], xt[hb:]], axis=1)

    return pl.pallas_call(
        body,
        grid=(nb,),
        in_specs=[pl.BlockSpec((D, TBLOCK), lambda i: (0, i))],
        out_specs=pl.BlockSpec((hb, 2 * D), lambda i: (i, 0)),
        out_shape=jax.ShapeDtypeStruct((nb * hb, 2 * D), jnp.float32),
    )(ent_t)


def _make_transe(B, D):
    info = plsc.get_sparse_core_info()
    NC, NS = info.num_cores, info.num_subcores
    NW = NC * NS
    per_w = B // NW
    n_chunks = per_w // CHUNK
    D2 = 2 * D
    assert per_w % CHUNK == 0 and D % LANES == 0

    mesh = plsc.VectorSubcoreMesh(core_axis_name="c", subcore_axis_name="s")

    row_buf = pltpu.VMEM((CHUNK, D2), jnp.float32)
    idx_buf = pltpu.VMEM((per_w,), jnp.int32)

    @functools.partial(
        pl.kernel,
        mesh=mesh,
        compiler_params=pltpu.CompilerParams(needs_layout_passes=False),
        out_type=(
            jax.ShapeDtypeStruct((B,), jnp.float32),
            jax.ShapeDtypeStruct((B,), jnp.float32),
        ),
        scratch_types=[
            idx_buf, idx_buf, idx_buf, idx_buf, idx_buf,
            idx_buf, idx_buf, idx_buf, idx_buf, idx_buf,
            row_buf, row_buf, row_buf, row_buf, row_buf,
            row_buf, row_buf, row_buf, row_buf, row_buf,
            pltpu.VMEM((per_w,), jnp.float32),
            pltpu.VMEM((per_w,), jnp.float32),
            pltpu.SemaphoreType.DMA,
            pltpu.SemaphoreType.DMA,
            pltpu.SemaphoreType.DMA,
        ],
    )
    def transe(ph_idx, pt_idx, nh_idx, nt_idx, r_idx, ent2, rel2,
               pos_out, neg_out,
               oph, opt, onh, ont, orl,
               tph, tpt, tnh, tnt, trl,
               ph0, pt0, nh0, nt0, rr0,
               ph1, pt1, nh1, nt1, rr1,
               po, no, sem_i, sem0, sem1):
        wid = lax.axis_index("s") * NC + lax.axis_index("c")
        base_w = wid * per_w
        offs_b = (oph, opt, onh, ont, orl)
        tids = (tph, tpt, tnh, tnt, trl)
        bufs = ((ph0, pt0, nh0, nt0, rr0), (ph1, pt1, nh1, nt1, rr1))
        sems = (sem0, sem1)

        idx_cps = [
            pltpu.async_copy(src.at[pl.ds(base_w, per_w)], dst, sem_i)
            for src, dst in zip(
                (ph_idx, pt_idx, nh_idx, nt_idx, r_idx), offs_b)
        ]
        for cp in idx_cps:
            cp.wait()

        hb_shift = jnp.int32(TBLOCK.bit_length() - 2)
        hb_mask = jnp.int32(TBLOCK // 2 - 1)
        dd = jnp.int32(D)

        def map_body(i, _):
            sl = pl.ds(i * LANES, LANES)
            for off, tid in zip(offs_b[:4], tids[:4]):
                e = off[sl]
                b = lax.shift_right_logical(e, hb_shift + 1)
                h = lax.shift_right_logical(e, hb_shift) & jnp.int32(1)
                tid[sl] = lax.shift_left(b, hb_shift) + (e & hb_mask)
                off[sl] = h * dd
            e = orl[sl]
            trl[sl] = lax.shift_right_logical(e, jnp.int32(1))
            orl[sl] = (e & jnp.int32(1)) * dd
            return 0

        lax.fori_loop(0, per_w // LANES, map_body, 0)

        def fire(c, par):
            sl = pl.ds(c * CHUNK, CHUNK)
            sem = sems[par]
            cps = []
            for tid, dst in zip(tids[:4], bufs[par][:4]):
                cps.append(pltpu.async_copy(ent2.at[tid.at[sl]], dst, sem))
            cps.append(pltpu.async_copy(rel2.at[trl.at[sl]], bufs[par][4], sem))
            return cps

        lane_ids = lax.iota(jnp.int32, LANES)
        in_flight = fire(0, 0)
        for c in range(n_chunks):
            par = c & 1
            for cp in in_flight:
                cp.wait()
            if c + 1 < n_chunks:
                in_flight = fire(c + 1, 1 - par)
            bset = bufs[par]
            out0 = c * CHUNK

            def group_body(g, _):
                row0 = g * LANES
                pvs = [off[pl.ds(out0 + row0, LANES)] for off in offs_b]
                pvec = jnp.zeros((LANES,), jnp.float32)
                nvec = jnp.zeros((LANES,), jnp.float32)
                for j in range(LANES):
                    r = row0 + j
                    offs = [pv[j] for pv in pvs]
                    pacc = jnp.zeros((LANES,), jnp.float32)
                    nacc = jnp.zeros((LANES,), jnp.float32)
                    for d in range(D // LANES):
                        hv = bset[0][r, pl.ds(offs[0] + d * LANES, LANES)]
                        tv = bset[1][r, pl.ds(offs[1] + d * LANES, LANES)]
                        nhv = bset[2][r, pl.ds(offs[2] + d * LANES, LANES)]
                        ntv = bset[3][r, pl.ds(offs[3] + d * LANES, LANES)]
                        rv = bset[4][r, pl.ds(offs[4] + d * LANES, LANES)]
                        pd = hv + rv - tv
                        nd = nhv + rv - ntv
                        pacc = pacc + pd * pd
                        nacc = nacc + nd * nd
                    jmask = lane_ids == j
                    pvec = jnp.where(jmask, jnp.sum(pacc), pvec)
                    nvec = jnp.where(jmask, jnp.sum(nacc), nvec)
                po[pl.ds(out0 + row0, LANES)] = _vec_sqrt(pvec)
                no[pl.ds(out0 + row0, LANES)] = _vec_sqrt(nvec)
                return 0

            lax.fori_loop(0, CHUNK // LANES, group_body, 0)

        pltpu.sync_copy(po, pos_out.at[pl.ds(base_w, per_w)])
        pltpu.sync_copy(no, neg_out.at[pl.ds(base_w, per_w)])

    return transe


def kernel(pos_edge_index, edge_type, neg_edge_index, entity_embeddings,
           relation_embeddings):
    B = pos_edge_index.shape[1]
    E, D = entity_embeddings.shape
    R = relation_embeddings.shape[0]
    ent2 = _transpose_pack(entity_embeddings.T)
    rel2 = relation_embeddings.reshape(R // 2, 2 * D)
    fn = _make_transe(B, D)
    return fn(pos_edge_index[0], pos_edge_index[1],
              neg_edge_index[0], neg_edge_index[1], edge_type, ent2, rel2)


# speedup vs baseline: 1.8843x; 1.0549x over previous
import functools

import jax
import jax.numpy as jnp
from jax import lax
from jax.experimental import pallas as pl
from jax.experimental.pallas import tpu as pltpu
from jax.experimental.pallas import tpu_sc as plsc

LANES = 16
CHUNK = 64
TBLOCK = 32768


def _vec_sqrt(x):
    xg = jnp.maximum(x, jnp.float32(1e-35))
    i = lax.bitcast_convert_type(xg, jnp.int32)
    i = jnp.int32(0x5F3759DF) - lax.shift_right_logical(i, jnp.int32(1))
    y = lax.bitcast_convert_type(i, jnp.float32)
    half = jnp.float32(0.5) * xg
    for _ in range(3):
        y = y * (jnp.float32(1.5) - half * y * y)
    return x * y


def _transpose_pack(ent_t):
    D, E = ent_t.shape
    nb = -(-E // TBLOCK)
    hb = TBLOCK // 2

    def body(in_ref, out_ref):
        lanes = jnp.arange(D, dtype=jnp.int32)
        ident = (lanes[:, None] == lanes[None, :]).astype(jnp.float32)
        xt = lax.dot_general(
            in_ref[...], ident, (((0,), (0,)), ((), ())),
            preferred_element_type=jnp.float32)
        out_ref[...] = jnp.concatenate([xt[:hb], xt[hb:]], axis=1)

    return pl.pallas_call(
        body,
        grid=(nb,),
        in_specs=[pl.BlockSpec((D, TBLOCK), lambda i: (0, i))],
        out_specs=pl.BlockSpec((hb, 2 * D), lambda i: (i, 0)),
        out_shape=jax.ShapeDtypeStruct((nb * hb, 2 * D), jnp.float32),
    )(ent_t)


def _make_transe(B, D):
    info = plsc.get_sparse_core_info()
    NC, NS = info.num_cores, info.num_subcores
    NW = NC * NS
    per_w = B // NW
    n_chunks = per_w // CHUNK
    D2 = 2 * D
    assert per_w % CHUNK == 0 and D % LANES == 0

    mesh = plsc.VectorSubcoreMesh(core_axis_name="c", subcore_axis_name="s")

    row_buf = pltpu.VMEM((CHUNK, D2), jnp.float32)
    idx_buf = pltpu.VMEM((per_w,), jnp.int32)

    @functools.partial(
        pl.kernel,
        mesh=mesh,
        compiler_params=pltpu.CompilerParams(needs_layout_passes=False),
        out_type=(
            jax.ShapeDtypeStruct((B,), jnp.float32),
            jax.ShapeDtypeStruct((B,), jnp.float32),
        ),
        scratch_types=[
            idx_buf, idx_buf, idx_buf, idx_buf, idx_buf,
            idx_buf, idx_buf, idx_buf, idx_buf, idx_buf,
            row_buf, row_buf, row_buf, row_buf, row_buf,
            row_buf, row_buf, row_buf, row_buf, row_buf,
            pltpu.VMEM((per_w,), jnp.float32),
            pltpu.VMEM((per_w,), jnp.float32),
            pltpu.SemaphoreType.DMA,
            pltpu.SemaphoreType.DMA,
            pltpu.SemaphoreType.DMA,
        ],
    )
    def transe(ph_idx, pt_idx, nh_idx, nt_idx, r_idx, ent2, rel2,
               pos_out, neg_out,
               oph, opt, onh, ont, orl,
               tph, tpt, tnh, tnt, trl,
               ph0, pt0, nh0, nt0, rr0,
               ph1, pt1, nh1, nt1, rr1,
               po, no, sem_i, sem0, sem1):
        wid = lax.axis_index("s") * NC + lax.axis_index("c")
        base_w = wid * per_w
        offs_b = (oph, opt, onh, ont, orl)
        tids = (tph, tpt, tnh, tnt, trl)
        bufs = ((ph0, pt0, nh0, nt0, rr0), (ph1, pt1, nh1, nt1, rr1))
        sems = (sem0, sem1)

        idx_cps = [
            pltpu.async_copy(src.at[pl.ds(base_w, per_w)], dst, sem_i)
            for src, dst in zip(
                (ph_idx, pt_idx, nh_idx, nt_idx, r_idx), offs_b)
        ]
        for cp in idx_cps:
            cp.wait()

        hb_shift = jnp.int32(TBLOCK.bit_length() - 2)
        hb_mask = jnp.int32(TBLOCK // 2 - 1)
        dd = jnp.int32(D)

        def map_body(i, _):
            sl = pl.ds(i * LANES, LANES)
            for off, tid in zip(offs_b[:4], tids[:4]):
                e = off[sl]
                b = lax.shift_right_logical(e, hb_shift + 1)
                h = lax.shift_right_logical(e, hb_shift) & jnp.int32(1)
                tid[sl] = lax.shift_left(b, hb_shift) + (e & hb_mask)
                off[sl] = h * dd
            e = orl[sl]
            trl[sl] = lax.shift_right_logical(e, jnp.int32(1))
            orl[sl] = (e & jnp.int32(1)) * dd
            return 0

        lax.fori_loop(0, per_w // LANES, map_body, 0)

        def fire(c, par):
            sl = pl.ds(c * CHUNK, CHUNK)
            sem = sems[par]
            cps = []
            for tid, dst in zip(tids[:4], bufs[par][:4]):
                cps.append(pltpu.async_copy(ent2.at[tid.at[sl]], dst, sem))
            cps.append(pltpu.async_copy(rel2.at[trl.at[sl]], bufs[par][4], sem))
            return cps

        lane_ids = lax.iota(jnp.int32, LANES)
        in_flight = fire(0, 0)
        for c in range(n_chunks):
            par = c & 1
            for cp in in_flight:
                cp.wait()
            if c + 1 < n_chunks:
                in_flight = fire(c + 1, 1 - par)
            bset = bufs[par]
            out0 = c * CHUNK

            def group_body(g, _):
                row0 = g * LANES
                pvs = [off[pl.ds(out0 + row0, LANES)] for off in offs_b]
                pvec = jnp.zeros((LANES,), jnp.float32)
                nvec = jnp.zeros((LANES,), jnp.float32)
                for j in range(LANES):
                    r = row0 + j
                    offs = [pv[j] for pv in pvs]
                    pacc = jnp.zeros((LANES,), jnp.float32)
                    nacc = jnp.zeros((LANES,), jnp.float32)
                    for d in range(D // LANES):
                        hv = bset[0][r, pl.ds(offs[0] + d * LANES, LANES)]
                        tv = bset[1][r, pl.ds(offs[1] + d * LANES, LANES)]
                        nhv = bset[2][r, pl.ds(offs[2] + d * LANES, LANES)]
                        ntv = bset[3][r, pl.ds(offs[3] + d * LANES, LANES)]
                        rv = bset[4][r, pl.ds(offs[4] + d * LANES, LANES)]
                        pd = hv + rv - tv
                        nd = nhv + rv - ntv
                        pacc = pacc + pd * pd
                        nacc = nacc + nd * nd
                    jmask = lane_ids == j
                    pvec = jnp.where(jmask, jnp.sum(pacc), pvec)
                    nvec = jnp.where(jmask, jnp.sum(nacc), nvec)
                po[pl.ds(out0 + row0, LANES)] = _vec_sqrt(pvec)
                no[pl.ds(out0 + row0, LANES)] = _vec_sqrt(nvec)
                return 0

            lax.fori_loop(0, CHUNK // LANES, group_body, 0)

        pltpu.sync_copy(po, pos_out.at[pl.ds(base_w, per_w)])
        pltpu.sync_copy(no, neg_out.at[pl.ds(base_w, per_w)])

    return transe


def kernel(pos_edge_index, edge_type, neg_edge_index, entity_embeddings,
           relation_embeddings):
    B = pos_edge_index.shape[1]
    E, D = entity_embeddings.shape
    R = relation_embeddings.shape[0]
    ent2 = _transpose_pack(entity_embeddings.T)
    rel2 = relation_embeddings.reshape(R // 2, 2 * D)
    fn = _make_transe(B, D)
    return fn(pos_edge_index[0], pos_edge_index[1],
              neg_edge_index[0], neg_edge_index[1], edge_type, ent2, rel2)
